# Initial kernel scaffold; baseline (speedup 1.0000x reference)
import jax
import jax.numpy as jnp
from jax.experimental import pallas as pl


def kernel(i, j, xij, w, w_, b, b_):
    raise NotImplementedError("write your pallas kernel here")


# speedup vs baseline: 1.4171x; 1.4171x over previous
import functools

import jax
import jax.numpy as jnp
from jax import lax
from jax.experimental import pallas as pl
from jax.experimental.pallas import tpu as pltpu
from jax.experimental.pallas import tpu_sc as plsc

_V = 1000000
_E = 32
_N = 1048576
_XMAX = 100.0
_ALPHA = 0.75

_NC, _NS, _L = 2, 16, 16
_NW = _NC * _NS
_P = _N // _NW
_C = 256
_G = _C // 128
_NCHUNK = _P // _C
_ROWS = _N // 128
_RPT = _ROWS // _NW


def _pre_body(x_ref, y_ref, f_ref):
    x = x_ref[...]
    y_ref[...] = jnp.log(x)
    f_ref[...] = jnp.minimum(jnp.exp(_ALPHA * jnp.log(x * (1.0 / _XMAX))), 1.0)


def _pre(x2):
    blk = pl.BlockSpec((1024, 128), lambda r: (r, 0))
    return pl.pallas_call(
        _pre_body,
        grid=(_ROWS // 1024,),
        in_specs=[blk],
        out_specs=[blk, blk],
        out_shape=[
            jax.ShapeDtypeStruct((_ROWS, 128), jnp.float32),
            jax.ShapeDtypeStruct((_ROWS, 128), jnp.float32),
        ],
    )(x2)


def _post_body(p_ref, o_ref):
    o_ref[0, 0] = jnp.sum(p_ref[...]) * (1.0 / _N)


def _post(parts):
    return pl.pallas_call(
        _post_body,
        in_specs=[pl.BlockSpec(memory_space=pltpu.VMEM)],
        out_specs=pl.BlockSpec(memory_space=pltpu.SMEM),
        out_shape=jax.ShapeDtypeStruct((1, 1), jnp.float32),
    )(parts)


def _sc_body(i2, j2, y2, f2, w, w_, b, b_, out,
             ii, jj,
             wi0, wj0, bi0, bj0, yb0, fb0,
             wi1, wj1, bi1, bj1, yb1, fb1,
             accv, sem0, sem1):
    cid = lax.axis_index("c")
    sid = lax.axis_index("s")
    wid = sid * _NC + cid

    pltpu.sync_copy(i2.at[pl.ds(wid * _RPT, _RPT)], ii)
    pltpu.sync_copy(j2.at[pl.ds(wid * _RPT, _RPT)], jj)

    def copies(t, wiB, wjB, biB, bjB, ybB, fbB, sem):
        r = t * _G
        hr = wid * _RPT + t * _G
        ops = [
            (y2.at[pl.ds(hr, _G)], ybB),
            (f2.at[pl.ds(hr, _G)], fbB),
        ]
        for g in range(_G):
            dst = pl.ds(g * 128, 128)
            ops.append((w.at[ii.at[r + g]], wiB.at[dst]))
            ops.append((w_.at[jj.at[r + g]], wjB.at[dst]))
            ops.append((b.at[ii.at[r + g]], biB.at[dst]))
            ops.append((b_.at[jj.at[r + g]], bjB.at[dst]))
        return [(s, d, sem) for (s, d) in ops]

    def fire(t, *buf):
        for s, d, sem in copies(t, *buf):
            pltpu.async_copy(s, d, sem)

    def drain(t, *buf):
        for s, d, sem in copies(t, *buf):
            pltpu.make_async_copy(s, d, sem).wait()

    dimv = [jnp.full((_L,), d, jnp.int32) for d in range(_E)]

    def compute(wiB, wjB, biB, bjB, ybB, fbB, acc):
        def blk(q, acc):
            rows = q * _L + lax.iota(jnp.int32, _L)
            s = plsc.load_gather(wiB, [rows, dimv[0]]) * \
                plsc.load_gather(wjB, [rows, dimv[0]])
            for d in range(1, _E):
                s = s + plsc.load_gather(wiB, [rows, dimv[d]]) * \
                        plsc.load_gather(wjB, [rows, dimv[d]])
            col = (q % 8) * _L
            row = q // 8
            e = s + biB[pl.ds(q * _L, _L)] + bjB[pl.ds(q * _L, _L)] \
                - ybB[row, pl.ds(col, _L)]
            return acc + fbB[row, pl.ds(col, _L)] * e * e
        return lax.fori_loop(0, _C // _L, blk, acc)

    buf0 = (wi0, wj0, bi0, bj0, yb0, fb0, sem0)
    buf1 = (wi1, wj1, bi1, bj1, yb1, fb1, sem1)

    fire(0, *buf0)

    def outer(k, acc):
        t0 = 2 * k
        fire(t0 + 1, *buf1)
        drain(t0, *buf0)
        acc = compute(*buf0[:6], acc)

        @pl.when(k < _NCHUNK // 2 - 1)
        def _():
            fire(t0 + 2, *buf0)

        drain(t0 + 1, *buf1)
        acc = compute(*buf1[:6], acc)
        return acc

    acc = lax.fori_loop(0, _NCHUNK // 2, outer, jnp.zeros((_L,), jnp.float32))
    accv[...] = acc
    pltpu.sync_copy(accv, out.at[wid])


def _sc(i2, j2, y2, f2, w, w_, b, b_):
    mesh = plsc.VectorSubcoreMesh(
        core_axis_name="c", subcore_axis_name="s",
        num_cores=_NC, num_subcores=_NS)
    kfn = pl.kernel(
        _sc_body,
        out_type=jax.ShapeDtypeStruct((_NW, _L), jnp.float32),
        mesh=mesh,
        compiler_params=pltpu.CompilerParams(
            needs_layout_passes=False, use_tc_tiling_on_sc=False),
        scratch_types=[
            pltpu.VMEM((_RPT, 128), jnp.int32),
            pltpu.VMEM((_RPT, 128), jnp.int32),
            pltpu.VMEM((_C, _E), jnp.float32),
            pltpu.VMEM((_C, _E), jnp.float32),
            pltpu.VMEM((_C,), jnp.float32),
            pltpu.VMEM((_C,), jnp.float32),
            pltpu.VMEM((_G, 128), jnp.float32),
            pltpu.VMEM((_G, 128), jnp.float32),
            pltpu.VMEM((_C, _E), jnp.float32),
            pltpu.VMEM((_C, _E), jnp.float32),
            pltpu.VMEM((_C,), jnp.float32),
            pltpu.VMEM((_C,), jnp.float32),
            pltpu.VMEM((_G, 128), jnp.float32),
            pltpu.VMEM((_G, 128), jnp.float32),
            pltpu.VMEM((_L,), jnp.float32),
            pltpu.SemaphoreType.DMA,
            pltpu.SemaphoreType.DMA,
        ],
    )
    return kfn(i2, j2, y2, f2, w, w_, b, b_)


def kernel(i, j, xij, w, w_, b, b_):
    i2 = i.reshape(_ROWS, 128)
    j2 = j.reshape(_ROWS, 128)
    x2 = xij.reshape(_ROWS, 128)
    y2, f2 = _pre(x2)
    parts = _sc(i2, j2, y2, f2, w, w_, b, b_)
    return _post(parts.reshape(4, 128))[0, 0]


# speedup vs baseline: 1.4200x; 1.0021x over previous
import jax
import jax.numpy as jnp
from jax import lax
from jax.experimental import pallas as pl
from jax.experimental.pallas import tpu as pltpu
from jax.experimental.pallas import tpu_sc as plsc

_V = 1000000
_E = 32
_N = 1048576
_XMAX = 100.0
_ALPHA = 0.75

_NC, _NS, _L = 2, 16, 16
_NW = _NC * _NS
_P = _N // _NW
_C = 512
_NCHUNK = _P // _C


def _pre_body(x_ref, y_ref, f_ref):
    x = x_ref[...]
    y_ref[...] = jnp.log(x)
    f_ref[...] = jnp.minimum(jnp.exp(_ALPHA * jnp.log(x * (1.0 / _XMAX))), 1.0)


def _pre(x2):
    blk = pl.BlockSpec((1024, 128), lambda r: (r, 0))
    return pl.pallas_call(
        _pre_body,
        grid=(_N // 128 // 1024,),
        in_specs=[blk],
        out_specs=[blk, blk],
        out_shape=[
            jax.ShapeDtypeStruct((_N // 128, 128), jnp.float32),
            jax.ShapeDtypeStruct((_N // 128, 128), jnp.float32),
        ],
    )(x2)


def _post_body(p_ref, o_ref):
    o_ref[0, 0] = jnp.sum(p_ref[...]) * (1.0 / _N)


def _post(parts):
    return pl.pallas_call(
        _post_body,
        in_specs=[pl.BlockSpec(memory_space=pltpu.VMEM)],
        out_specs=pl.BlockSpec(memory_space=pltpu.SMEM),
        out_shape=jax.ShapeDtypeStruct((1, 1), jnp.float32),
    )(parts)


def _sc_body(iv, jv, yv, fv, w, w_, b, b_, out,
             ii0, jj0, ii1, jj1,
             wi0, wj0, bi0, bj0, yb0, fb0,
             wi1, wj1, bi1, bj1, yb1, fb1,
             accv, semL0, semL1, semG0, semG1):
    cid = lax.axis_index("c")
    sid = lax.axis_index("s")
    wid = sid * _NC + cid
    base = wid * _P

    idx0 = (ii0, jj0, semL0)
    idx1 = (ii1, jj1, semL1)
    row0 = (wi0, wj0, bi0, bj0, yb0, fb0, semG0)
    row1 = (wi1, wj1, bi1, bj1, yb1, fb1, semG1)

    def l_copies(t, ib):
        ii, jj, sem = ib
        o = base + t * _C
        return [(iv.at[pl.ds(o, _C)], ii, sem),
                (jv.at[pl.ds(o, _C)], jj, sem)]

    def g_copies(t, ib, rb):
        ii, jj, _ = ib
        wi, wj, bi, bj, yb, fb, sem = rb
        o = base + t * _C
        return [(w.at[ii], wi, sem),
                (w_.at[jj], wj, sem),
                (b.at[ii], bi, sem),
                (b_.at[jj], bj, sem),
                (yv.at[pl.ds(o, _C)], yb, sem),
                (fv.at[pl.ds(o, _C)], fb, sem)]

    def fire(ops):
        for s, d, sem in ops:
            pltpu.async_copy(s, d, sem)

    def drain(ops):
        for s, d, sem in ops:
            pltpu.make_async_copy(s, d, sem).wait()

    dimv = [jnp.full((_L,), d, jnp.int32) for d in range(_E)]

    def compute(rb, acc):
        wi, wj, bi, bj, yb, fb, _ = rb

        def blk(q, acc):
            rows = q * _L + lax.iota(jnp.int32, _L)
            s = plsc.load_gather(wi, [rows, dimv[0]]) * \
                plsc.load_gather(wj, [rows, dimv[0]])
            for d in range(1, _E):
                s = s + plsc.load_gather(wi, [rows, dimv[d]]) * \
                        plsc.load_gather(wj, [rows, dimv[d]])
            sl = pl.ds(q * _L, _L)
            e = s + bi[sl] + bj[sl] - yb[sl]
            return acc + fb[sl] * e * e
        return lax.fori_loop(0, _C // _L, blk, acc)

    fire(l_copies(0, idx0))
    drain(l_copies(0, idx0))
    fire(g_copies(0, idx0, row0))
    fire(l_copies(1, idx1))

    def outer(k, acc):
        t = 2 * k
        drain(l_copies(t + 1, idx1))
        fire(g_copies(t + 1, idx1, row1))
        drain(g_copies(t, idx0, row0))

        @pl.when(t + 2 < _NCHUNK)
        def _():
            fire(l_copies(t + 2, idx0))
        acc = compute(row0, acc)

        @pl.when(t + 2 < _NCHUNK)
        def _():
            drain(l_copies(t + 2, idx0))
            fire(g_copies(t + 2, idx0, row0))

        drain(g_copies(t + 1, idx1, row1))

        @pl.when(t + 3 < _NCHUNK)
        def _():
            fire(l_copies(t + 3, idx1))
        acc = compute(row1, acc)
        return acc

    acc = lax.fori_loop(0, _NCHUNK // 2, outer, jnp.zeros((_L,), jnp.float32))
    accv[...] = acc
    pltpu.sync_copy(accv, out.at[wid])


def _sc(iv, jv, yv, fv, w, w_, b, b_):
    mesh = plsc.VectorSubcoreMesh(
        core_axis_name="c", subcore_axis_name="s",
        num_cores=_NC, num_subcores=_NS)
    kfn = pl.kernel(
        _sc_body,
        out_type=jax.ShapeDtypeStruct((_NW, _L), jnp.float32),
        mesh=mesh,
        compiler_params=pltpu.CompilerParams(
            needs_layout_passes=False, use_tc_tiling_on_sc=False),
        scratch_types=[
            pltpu.VMEM((_C,), jnp.int32),
            pltpu.VMEM((_C,), jnp.int32),
            pltpu.VMEM((_C,), jnp.int32),
            pltpu.VMEM((_C,), jnp.int32),
            pltpu.VMEM((_C, _E), jnp.float32),
            pltpu.VMEM((_C, _E), jnp.float32),
            pltpu.VMEM((_C,), jnp.float32),
            pltpu.VMEM((_C,), jnp.float32),
            pltpu.VMEM((_C,), jnp.float32),
            pltpu.VMEM((_C,), jnp.float32),
            pltpu.VMEM((_C, _E), jnp.float32),
            pltpu.VMEM((_C, _E), jnp.float32),
            pltpu.VMEM((_C,), jnp.float32),
            pltpu.VMEM((_C,), jnp.float32),
            pltpu.VMEM((_C,), jnp.float32),
            pltpu.VMEM((_C,), jnp.float32),
            pltpu.VMEM((_L,), jnp.float32),
            pltpu.SemaphoreType.DMA,
            pltpu.SemaphoreType.DMA,
            pltpu.SemaphoreType.DMA,
            pltpu.SemaphoreType.DMA,
        ],
    )
    return kfn(iv, jv, yv, fv, w, w_, b, b_)


def kernel(i, j, xij, w, w_, b, b_):
    x2 = xij.reshape(_N // 128, 128)
    y2, f2 = _pre(x2)
    parts = _sc(i, j, y2.reshape(_N), f2.reshape(_N), w, w_, b, b_)
    return _post(parts.reshape(4, 128))[0, 0]


# speedup vs baseline: 1.4211x; 1.0008x over previous
import jax
import jax.numpy as jnp
from jax import lax
from jax.experimental import pallas as pl
from jax.experimental.pallas import tpu as pltpu
from jax.experimental.pallas import tpu_sc as plsc

_V = 1000000
_E = 32
_N = 1048576
_XMAX = 100.0
_ALPHA = 0.75

_NC, _NS, _L = 2, 16, 16
_NW = _NC * _NS
_P = _N // _NW
_C = 512
_NCHUNK = _P // _C


def _pre_body(x_ref, y_ref, f_ref):
    x = x_ref[...]
    y_ref[...] = jnp.log(x)
    f_ref[...] = jnp.minimum(jnp.exp(_ALPHA * jnp.log(x * (1.0 / _XMAX))), 1.0)


def _pre(x):
    blk = pl.BlockSpec((131072,), lambda r: (r,))
    return pl.pallas_call(
        _pre_body,
        grid=(_N // 131072,),
        in_specs=[blk],
        out_specs=[blk, blk],
        out_shape=[
            jax.ShapeDtypeStruct((_N,), jnp.float32),
            jax.ShapeDtypeStruct((_N,), jnp.float32),
        ],
    )(x)


def _post_body(p_ref, o_ref):
    o_ref[0, 0] = jnp.sum(p_ref[...]) * (1.0 / _N)


def _post(parts):
    return pl.pallas_call(
        _post_body,
        in_specs=[pl.BlockSpec(memory_space=pltpu.VMEM)],
        out_specs=pl.BlockSpec(memory_space=pltpu.SMEM),
        out_shape=jax.ShapeDtypeStruct((1, 1), jnp.float32),
    )(parts)


def _sc_body(iv, jv, yv, fv, w, w_, b, b_, out,
             ii0, jj0, ii1, jj1,
             wi0, wj0, bi0, bj0, yb0, fb0,
             wi1, wj1, bi1, bj1, yb1, fb1,
             accv, semL0, semL1, semG0, semG1):
    cid = lax.axis_index("c")
    sid = lax.axis_index("s")
    wid = sid * _NC + cid
    base = wid * _P

    idx0 = (ii0, jj0, semL0)
    idx1 = (ii1, jj1, semL1)
    row0 = (wi0, wj0, bi0, bj0, yb0, fb0, semG0)
    row1 = (wi1, wj1, bi1, bj1, yb1, fb1, semG1)

    def l_copies(t, ib):
        ii, jj, sem = ib
        o = base + t * _C
        return [(iv.at[pl.ds(o, _C)], ii, sem),
                (jv.at[pl.ds(o, _C)], jj, sem)]

    def g_copies(t, ib, rb):
        ii, jj, _ = ib
        wi, wj, bi, bj, yb, fb, sem = rb
        o = base + t * _C
        return [(w.at[ii], wi, sem),
                (w_.at[jj], wj, sem),
                (b.at[ii], bi, sem),
                (b_.at[jj], bj, sem),
                (yv.at[pl.ds(o, _C)], yb, sem),
                (fv.at[pl.ds(o, _C)], fb, sem)]

    def fire(ops):
        for s, d, sem in ops:
            pltpu.async_copy(s, d, sem)

    def drain(ops):
        for s, d, sem in ops:
            pltpu.make_async_copy(s, d, sem).wait()

    dimv = [jnp.full((_L,), d, jnp.int32) for d in range(_E)]

    def compute(rb, acc):
        wi, wj, bi, bj, yb, fb, _ = rb

        def blk(q, acc):
            rows = q * _L + lax.iota(jnp.int32, _L)
            s = plsc.load_gather(wi, [rows, dimv[0]]) * \
                plsc.load_gather(wj, [rows, dimv[0]])
            for d in range(1, _E):
                s = s + plsc.load_gather(wi, [rows, dimv[d]]) * \
                        plsc.load_gather(wj, [rows, dimv[d]])
            sl = pl.ds(q * _L, _L)
            e = s + bi[sl] + bj[sl] - yb[sl]
            return acc + fb[sl] * e * e
        return lax.fori_loop(0, _C // _L, blk, acc)

    fire(l_copies(0, idx0))
    drain(l_copies(0, idx0))
    fire(g_copies(0, idx0, row0))
    fire(l_copies(1, idx1))

    def outer(k, acc):
        t = 2 * k
        drain(l_copies(t + 1, idx1))
        fire(g_copies(t + 1, idx1, row1))
        drain(g_copies(t, idx0, row0))

        @pl.when(t + 2 < _NCHUNK)
        def _():
            fire(l_copies(t + 2, idx0))
        acc = compute(row0, acc)

        @pl.when(t + 2 < _NCHUNK)
        def _():
            drain(l_copies(t + 2, idx0))
            fire(g_copies(t + 2, idx0, row0))

        drain(g_copies(t + 1, idx1, row1))

        @pl.when(t + 3 < _NCHUNK)
        def _():
            fire(l_copies(t + 3, idx1))
        acc = compute(row1, acc)
        return acc

    acc = lax.fori_loop(0, _NCHUNK // 2, outer, jnp.zeros((_L,), jnp.float32))
    accv[...] = acc
    pltpu.sync_copy(accv, out.at[wid])


def _sc(iv, jv, yv, fv, w, w_, b, b_):
    mesh = plsc.VectorSubcoreMesh(
        core_axis_name="c", subcore_axis_name="s",
        num_cores=_NC, num_subcores=_NS)
    kfn = pl.kernel(
        _sc_body,
        out_type=jax.ShapeDtypeStruct((_NW, _L), jnp.float32),
        mesh=mesh,
        compiler_params=pltpu.CompilerParams(
            needs_layout_passes=False, use_tc_tiling_on_sc=False),
        scratch_types=[
            pltpu.VMEM((_C,), jnp.int32),
            pltpu.VMEM((_C,), jnp.int32),
            pltpu.VMEM((_C,), jnp.int32),
            pltpu.VMEM((_C,), jnp.int32),
            pltpu.VMEM((_C, _E), jnp.float32),
            pltpu.VMEM((_C, _E), jnp.float32),
            pltpu.VMEM((_C,), jnp.float32),
            pltpu.VMEM((_C,), jnp.float32),
            pltpu.VMEM((_C,), jnp.float32),
            pltpu.VMEM((_C,), jnp.float32),
            pltpu.VMEM((_C, _E), jnp.float32),
            pltpu.VMEM((_C, _E), jnp.float32),
            pltpu.VMEM((_C,), jnp.float32),
            pltpu.VMEM((_C,), jnp.float32),
            pltpu.VMEM((_C,), jnp.float32),
            pltpu.VMEM((_C,), jnp.float32),
            pltpu.VMEM((_L,), jnp.float32),
            pltpu.SemaphoreType.DMA,
            pltpu.SemaphoreType.DMA,
            pltpu.SemaphoreType.DMA,
            pltpu.SemaphoreType.DMA,
        ],
    )
    return kfn(iv, jv, yv, fv, w, w_, b, b_)


def kernel(i, j, xij, w, w_, b, b_):
    yv, fv = _pre(xij)
    parts = _sc(i, j, yv, fv, w, w_, b, b_)
    return _post(parts.reshape(4, 128))[0, 0]


# speedup vs baseline: 2.0769x; 1.4614x over previous
import jax
import jax.numpy as jnp
from jax import lax
from jax.experimental import pallas as pl
from jax.experimental.pallas import tpu as pltpu
from jax.experimental.pallas import tpu_sc as plsc

_V = 1000000
_E = 32
_N = 1048576
_XMAX = 100.0
_ALPHA = 0.75

_NC, _NS, _L = 2, 16, 16
_NW = _NC * _NS
_P = _N // _NW
_C = 512
_NCHUNK = _P // _C


def _pre_body(x_ref, y_ref, f_ref):
    x = x_ref[...]
    y_ref[...] = jnp.log(x)
    f_ref[...] = jnp.minimum(jnp.exp(_ALPHA * jnp.log(x * (1.0 / _XMAX))), 1.0)


def _pre(x):
    blk = pl.BlockSpec((131072,), lambda r: (r,))
    return pl.pallas_call(
        _pre_body,
        grid=(_N // 131072,),
        in_specs=[blk],
        out_specs=[blk, blk],
        out_shape=[
            jax.ShapeDtypeStruct((_N,), jnp.float32),
            jax.ShapeDtypeStruct((_N,), jnp.float32),
        ],
    )(x)


def _post_body(p_ref, o_ref):
    o_ref[0, 0] = jnp.sum(p_ref[...]) * (1.0 / _N)


def _post(parts):
    return pl.pallas_call(
        _post_body,
        in_specs=[pl.BlockSpec(memory_space=pltpu.VMEM)],
        out_specs=pl.BlockSpec(memory_space=pltpu.SMEM),
        out_shape=jax.ShapeDtypeStruct((1, 1), jnp.float32),
    )(parts)


def _sc_body(iv, jv, yv, fv, w, w_, b, b_, out,
             ii0, jj0, ii1, jj1,
             wi0, wj0, bi0, bj0, yb0, fb0,
             wi1, wj1, bi1, bj1, yb1, fb1,
             accv, semL0, semL1, semG0, semG1):
    cid = lax.axis_index("c")
    sid = lax.axis_index("s")
    wid = sid * _NC + cid
    base = wid * _P

    idx0 = (ii0, jj0, semL0)
    idx1 = (ii1, jj1, semL1)
    row0 = (wi0, wj0, bi0, bj0, yb0, fb0, semG0)
    row1 = (wi1, wj1, bi1, bj1, yb1, fb1, semG1)

    def l_copies(t, ib):
        ii, jj, sem = ib
        o = base + t * _C
        return [(iv.at[pl.ds(o, _C)], ii, sem),
                (jv.at[pl.ds(o, _C)], jj, sem)]

    def g_copies(t, ib, rb):
        ii, jj, _ = ib
        wi, wj, bi, bj, yb, fb, sem = rb
        o = base + t * _C
        return [(w.at[ii], wi, sem),
                (yv.at[pl.ds(o, _C)], yb, sem),
                (fv.at[pl.ds(o, _C)], fb, sem)]

    def fire(ops):
        for s, d, sem in ops:
            pltpu.async_copy(s, d, sem)

    def drain(ops):
        for s, d, sem in ops:
            pltpu.make_async_copy(s, d, sem).wait()

    dimv = [jnp.full((_L,), d, jnp.int32) for d in range(_E)]

    def compute(rb, acc):
        wi, wj, bi, bj, yb, fb, _ = rb

        def blk(q, acc):
            rows = q * _L + lax.iota(jnp.int32, _L)
            s = plsc.load_gather(wi, [rows, dimv[0]]) * \
                plsc.load_gather(wi, [rows, dimv[0]])
            for d in range(1, _E):
                s = s + plsc.load_gather(wi, [rows, dimv[d]]) * \
                        plsc.load_gather(wi, [rows, dimv[d]])
            sl = pl.ds(q * _L, _L)
            e = s - yb[sl]
            return acc + fb[sl] * e * e
        return lax.fori_loop(0, _C // _L, blk, acc)

    fire(l_copies(0, idx0))
    drain(l_copies(0, idx0))
    fire(g_copies(0, idx0, row0))
    fire(l_copies(1, idx1))

    def outer(k, acc):
        t = 2 * k
        drain(l_copies(t + 1, idx1))
        fire(g_copies(t + 1, idx1, row1))
        drain(g_copies(t, idx0, row0))

        @pl.when(t + 2 < _NCHUNK)
        def _():
            fire(l_copies(t + 2, idx0))
        acc = compute(row0, acc)

        @pl.when(t + 2 < _NCHUNK)
        def _():
            drain(l_copies(t + 2, idx0))
            fire(g_copies(t + 2, idx0, row0))

        drain(g_copies(t + 1, idx1, row1))

        @pl.when(t + 3 < _NCHUNK)
        def _():
            fire(l_copies(t + 3, idx1))
        acc = compute(row1, acc)
        return acc

    acc = lax.fori_loop(0, _NCHUNK // 2, outer, jnp.zeros((_L,), jnp.float32))
    accv[...] = acc
    pltpu.sync_copy(accv, out.at[wid])


def _sc(iv, jv, yv, fv, w, w_, b, b_):
    mesh = plsc.VectorSubcoreMesh(
        core_axis_name="c", subcore_axis_name="s",
        num_cores=_NC, num_subcores=_NS)
    kfn = pl.kernel(
        _sc_body,
        out_type=jax.ShapeDtypeStruct((_NW, _L), jnp.float32),
        mesh=mesh,
        compiler_params=pltpu.CompilerParams(
            needs_layout_passes=False, use_tc_tiling_on_sc=False),
        scratch_types=[
            pltpu.VMEM((_C,), jnp.int32),
            pltpu.VMEM((_C,), jnp.int32),
            pltpu.VMEM((_C,), jnp.int32),
            pltpu.VMEM((_C,), jnp.int32),
            pltpu.VMEM((_C, _E), jnp.float32),
            pltpu.VMEM((_C, _E), jnp.float32),
            pltpu.VMEM((_C,), jnp.float32),
            pltpu.VMEM((_C,), jnp.float32),
            pltpu.VMEM((_C,), jnp.float32),
            pltpu.VMEM((_C,), jnp.float32),
            pltpu.VMEM((_C, _E), jnp.float32),
            pltpu.VMEM((_C, _E), jnp.float32),
            pltpu.VMEM((_C,), jnp.float32),
            pltpu.VMEM((_C,), jnp.float32),
            pltpu.VMEM((_C,), jnp.float32),
            pltpu.VMEM((_C,), jnp.float32),
            pltpu.VMEM((_L,), jnp.float32),
            pltpu.SemaphoreType.DMA,
            pltpu.SemaphoreType.DMA,
            pltpu.SemaphoreType.DMA,
            pltpu.SemaphoreType.DMA,
        ],
    )
    return kfn(iv, jv, yv, fv, w, w_, b, b_)


def kernel(i, j, xij, w, w_, b, b_):
    yv, fv = _pre(xij)
    parts = _sc(i, j, yv, fv, w, w_, b, b_)
    return _post(parts.reshape(4, 128))[0, 0]


# speedup vs baseline: 2.5332x; 1.2197x over previous
import jax
import jax.numpy as jnp
from jax import lax
from jax.experimental import pallas as pl
from jax.experimental.pallas import tpu as pltpu
from jax.experimental.pallas import tpu_sc as plsc

_V = 1000000
_E = 32
_N = 1048576
_XMAX = 100.0
_ALPHA = 0.75

_NC, _NS, _L = 2, 16, 16
_NW = _NC * _NS
_P = _N // _NW
_C = 512
_NCHUNK = _P // _C


def _pre_body(x_ref, y_ref, f_ref):
    x = x_ref[...]
    y_ref[...] = jnp.log(x)
    f_ref[...] = jnp.minimum(jnp.exp(_ALPHA * jnp.log(x * (1.0 / _XMAX))), 1.0)


def _pre(x):
    blk = pl.BlockSpec((131072,), lambda r: (r,))
    return pl.pallas_call(
        _pre_body,
        grid=(_N // 131072,),
        in_specs=[blk],
        out_specs=[blk, blk],
        out_shape=[
            jax.ShapeDtypeStruct((_N,), jnp.float32),
            jax.ShapeDtypeStruct((_N,), jnp.float32),
        ],
    )(x)


def _post_body(p_ref, o_ref):
    o_ref[0, 0] = jnp.sum(p_ref[...]) * (1.0 / _N)


def _post(parts):
    return pl.pallas_call(
        _post_body,
        in_specs=[pl.BlockSpec(memory_space=pltpu.VMEM)],
        out_specs=pl.BlockSpec(memory_space=pltpu.SMEM),
        out_shape=jax.ShapeDtypeStruct((1, 1), jnp.float32),
    )(parts)


def _sc_body(iv, jv, yv, fv, w, w_, b, b_, out,
             ii0, jj0, ii1, jj1,
             wi0, wj0, bi0, bj0, yb0, fb0,
             wi1, wj1, bi1, bj1, yb1, fb1,
             accv, semL0, semL1, semG0, semG1):
    cid = lax.axis_index("c")
    sid = lax.axis_index("s")
    wid = sid * _NC + cid
    base = wid * _P

    idx0 = (ii0, jj0, semL0)
    idx1 = (ii1, jj1, semL1)
    row0 = (wi0, wj0, bi0, bj0, yb0, fb0, semG0)
    row1 = (wi1, wj1, bi1, bj1, yb1, fb1, semG1)

    def l_copies(t, ib):
        ii, jj, sem = ib
        o = base + t * _C
        return [(iv.at[pl.ds(o, _C)], ii, sem),
                (jv.at[pl.ds(o, _C)], jj, sem)]

    def g_copies(t, ib, rb):
        ii, jj, _ = ib
        wi, wj, bi, bj, yb, fb, sem = rb
        o = base + t * _C
        return [(w.at[ii], wi, sem),
                (w_.at[jj], wj, sem),
                (b.at[ii], bi, sem),
                (b_.at[jj], bj, sem),
                (yv.at[pl.ds(o, _C)], yb, sem),
                (fv.at[pl.ds(o, _C)], fb, sem)]

    def fire(ops):
        for s, d, sem in ops:
            pltpu.async_copy(s, d, sem)

    def drain(ops):
        for s, d, sem in ops:
            pltpu.make_async_copy(s, d, sem).wait()

    dimv = [jnp.full((_L,), d, jnp.int32) for d in range(_E)]

    def compute(rb, acc):
        wi, wj, bi, bj, yb, fb, _ = rb

        def blk(q, acc):
            rows = q * _L + lax.iota(jnp.int32, _L)
            s = plsc.load_gather(wi, [rows, dimv[0]]) * \
                plsc.load_gather(wj, [rows, dimv[0]])
            sl = pl.ds(q * _L, _L)
            e = s + bi[sl] + bj[sl] - yb[sl]
            return acc + fb[sl] * e * e
        return lax.fori_loop(0, _C // _L, blk, acc)

    fire(l_copies(0, idx0))
    drain(l_copies(0, idx0))
    fire(g_copies(0, idx0, row0))
    fire(l_copies(1, idx1))

    def outer(k, acc):
        t = 2 * k
        drain(l_copies(t + 1, idx1))
        fire(g_copies(t + 1, idx1, row1))
        drain(g_copies(t, idx0, row0))

        @pl.when(t + 2 < _NCHUNK)
        def _():
            fire(l_copies(t + 2, idx0))
        acc = compute(row0, acc)

        @pl.when(t + 2 < _NCHUNK)
        def _():
            drain(l_copies(t + 2, idx0))
            fire(g_copies(t + 2, idx0, row0))

        drain(g_copies(t + 1, idx1, row1))

        @pl.when(t + 3 < _NCHUNK)
        def _():
            fire(l_copies(t + 3, idx1))
        acc = compute(row1, acc)
        return acc

    acc = lax.fori_loop(0, _NCHUNK // 2, outer, jnp.zeros((_L,), jnp.float32))
    accv[...] = acc
    pltpu.sync_copy(accv, out.at[wid])


def _sc(iv, jv, yv, fv, w, w_, b, b_):
    mesh = plsc.VectorSubcoreMesh(
        core_axis_name="c", subcore_axis_name="s",
        num_cores=_NC, num_subcores=_NS)
    kfn = pl.kernel(
        _sc_body,
        out_type=jax.ShapeDtypeStruct((_NW, _L), jnp.float32),
        mesh=mesh,
        compiler_params=pltpu.CompilerParams(
            needs_layout_passes=False, use_tc_tiling_on_sc=False),
        scratch_types=[
            pltpu.VMEM((_C,), jnp.int32),
            pltpu.VMEM((_C,), jnp.int32),
            pltpu.VMEM((_C,), jnp.int32),
            pltpu.VMEM((_C,), jnp.int32),
            pltpu.VMEM((_C, _E), jnp.float32),
            pltpu.VMEM((_C, _E), jnp.float32),
            pltpu.VMEM((_C,), jnp.float32),
            pltpu.VMEM((_C,), jnp.float32),
            pltpu.VMEM((_C,), jnp.float32),
            pltpu.VMEM((_C,), jnp.float32),
            pltpu.VMEM((_C, _E), jnp.float32),
            pltpu.VMEM((_C, _E), jnp.float32),
            pltpu.VMEM((_C,), jnp.float32),
            pltpu.VMEM((_C,), jnp.float32),
            pltpu.VMEM((_C,), jnp.float32),
            pltpu.VMEM((_C,), jnp.float32),
            pltpu.VMEM((_L,), jnp.float32),
            pltpu.SemaphoreType.DMA,
            pltpu.SemaphoreType.DMA,
            pltpu.SemaphoreType.DMA,
            pltpu.SemaphoreType.DMA,
        ],
    )
    return kfn(iv, jv, yv, fv, w, w_, b, b_)


def kernel(i, j, xij, w, w_, b, b_):
    yv, fv = _pre(xij)
    parts = _sc(i, j, yv, fv, w, w_, b, b_)
    return _post(parts.reshape(4, 128))[0, 0]


# speedup vs baseline: 2.5403x; 1.0028x over previous
import jax
import jax.numpy as jnp
from jax import lax
from jax.experimental import pallas as pl
from jax.experimental.pallas import tpu as pltpu
from jax.experimental.pallas import tpu_sc as plsc

_V = 1000000
_E = 32
_N = 1048576
_XMAX = 100.0
_ALPHA = 0.75

_NC, _NS, _L = 2, 16, 16
_NW = _NC * _NS
_P = _N // _NW
_C = 512
_NCHUNK = _P // _C


def _pre_body(x_ref, y_ref, f_ref):
    x = x_ref[...]
    y_ref[...] = jnp.log(x)
    f_ref[...] = jnp.minimum(jnp.exp(_ALPHA * jnp.log(x * (1.0 / _XMAX))), 1.0)


def _pre(x):
    blk = pl.BlockSpec((131072,), lambda r: (r,))
    return pl.pallas_call(
        _pre_body,
        grid=(_N // 131072,),
        in_specs=[blk],
        out_specs=[blk, blk],
        out_shape=[
            jax.ShapeDtypeStruct((_N,), jnp.float32),
            jax.ShapeDtypeStruct((_N,), jnp.float32),
        ],
    )(x)


def _post_body(p_ref, o_ref):
    o_ref[0, 0] = jnp.sum(p_ref[...]) * (1.0 / _N)


def _post(parts):
    return pl.pallas_call(
        _post_body,
        in_specs=[pl.BlockSpec(memory_space=pltpu.VMEM)],
        out_specs=pl.BlockSpec(memory_space=pltpu.SMEM),
        out_shape=jax.ShapeDtypeStruct((1, 1), jnp.float32),
    )(parts)


def _sc_body(iv, jv, yv, fv, w, w_, b, b_, out,
             ii0, jj0, ii1, jj1,
             wi0, wj0, bi0, bj0, yb0, fb0,
             wi1, wj1, bi1, bj1, yb1, fb1,
             accv, semL0, semL1, semG0, semG1):
    cid = lax.axis_index("c")
    sid = lax.axis_index("s")
    wid = sid * _NC + cid
    base = wid * _P

    idx0 = (ii0, jj0, semL0)
    idx1 = (ii1, jj1, semL1)
    row0 = (wi0, wj0, bi0, bj0, yb0, fb0, semG0)
    row1 = (wi1, wj1, bi1, bj1, yb1, fb1, semG1)

    def l_copies(t, ib):
        ii, jj, sem = ib
        o = base + t * _C
        return [(iv.at[pl.ds(o, _C)], ii, sem),
                (jv.at[pl.ds(o, _C)], jj, sem)]

    def g_copies(t, ib, rb):
        ii, jj, _ = ib
        wi, wj, bi, bj, yb, fb, sem = rb
        o = base + t * _C
        return [(w.at[ii], wi, sem),
                (w_.at[jj], wj, sem),
                (b.at[ii], bi, sem),
                (b_.at[jj], bj, sem),
                (yv.at[pl.ds(o, _C)], yb, sem),
                (fv.at[pl.ds(o, _C)], fb, sem)]

    def fire(ops):
        for s, d, sem in ops:
            pltpu.async_copy(s, d, sem)

    def drain(ops):
        for s, d, sem in ops:
            pltpu.make_async_copy(s, d, sem).wait()

    lane = lax.iota(jnp.int32, _L)
    dimv = [(lane + d) & (_E - 1) for d in range(_E)]

    def compute(rb, acc):
        wi, wj, bi, bj, yb, fb, _ = rb

        def blk(q, acc):
            rows = q * _L + lane
            s0 = plsc.load_gather(wi, [rows, dimv[0]]) * \
                 plsc.load_gather(wj, [rows, dimv[0]])
            s1 = plsc.load_gather(wi, [rows, dimv[1]]) * \
                 plsc.load_gather(wj, [rows, dimv[1]])
            s2 = plsc.load_gather(wi, [rows, dimv[2]]) * \
                 plsc.load_gather(wj, [rows, dimv[2]])
            s3 = plsc.load_gather(wi, [rows, dimv[3]]) * \
                 plsc.load_gather(wj, [rows, dimv[3]])
            for d in range(4, _E, 4):
                s0 = s0 + plsc.load_gather(wi, [rows, dimv[d]]) * \
                          plsc.load_gather(wj, [rows, dimv[d]])
                s1 = s1 + plsc.load_gather(wi, [rows, dimv[d + 1]]) * \
                          plsc.load_gather(wj, [rows, dimv[d + 1]])
                s2 = s2 + plsc.load_gather(wi, [rows, dimv[d + 2]]) * \
                          plsc.load_gather(wj, [rows, dimv[d + 2]])
                s3 = s3 + plsc.load_gather(wi, [rows, dimv[d + 3]]) * \
                          plsc.load_gather(wj, [rows, dimv[d + 3]])
            s = (s0 + s1) + (s2 + s3)
            sl = pl.ds(q * _L, _L)
            e = s + bi[sl] + bj[sl] - yb[sl]
            return acc + fb[sl] * e * e
        return lax.fori_loop(0, _C // _L, blk, acc)

    fire(l_copies(0, idx0))
    drain(l_copies(0, idx0))
    fire(g_copies(0, idx0, row0))
    fire(l_copies(1, idx1))

    def outer(k, acc):
        t = 2 * k
        drain(l_copies(t + 1, idx1))
        fire(g_copies(t + 1, idx1, row1))
        drain(g_copies(t, idx0, row0))

        @pl.when(t + 2 < _NCHUNK)
        def _():
            fire(l_copies(t + 2, idx0))
        acc = compute(row0, acc)

        @pl.when(t + 2 < _NCHUNK)
        def _():
            drain(l_copies(t + 2, idx0))
            fire(g_copies(t + 2, idx0, row0))

        drain(g_copies(t + 1, idx1, row1))

        @pl.when(t + 3 < _NCHUNK)
        def _():
            fire(l_copies(t + 3, idx1))
        acc = compute(row1, acc)
        return acc

    acc = lax.fori_loop(0, _NCHUNK // 2, outer, jnp.zeros((_L,), jnp.float32))
    accv[...] = acc
    pltpu.sync_copy(accv, out.at[wid])


def _sc(iv, jv, yv, fv, w, w_, b, b_):
    mesh = plsc.VectorSubcoreMesh(
        core_axis_name="c", subcore_axis_name="s",
        num_cores=_NC, num_subcores=_NS)
    kfn = pl.kernel(
        _sc_body,
        out_type=jax.ShapeDtypeStruct((_NW, _L), jnp.float32),
        mesh=mesh,
        compiler_params=pltpu.CompilerParams(
            needs_layout_passes=False, use_tc_tiling_on_sc=False),
        scratch_types=[
            pltpu.VMEM((_C,), jnp.int32),
            pltpu.VMEM((_C,), jnp.int32),
            pltpu.VMEM((_C,), jnp.int32),
            pltpu.VMEM((_C,), jnp.int32),
            pltpu.VMEM((_C, _E), jnp.float32),
            pltpu.VMEM((_C, _E), jnp.float32),
            pltpu.VMEM((_C,), jnp.float32),
            pltpu.VMEM((_C,), jnp.float32),
            pltpu.VMEM((_C,), jnp.float32),
            pltpu.VMEM((_C,), jnp.float32),
            pltpu.VMEM((_C, _E), jnp.float32),
            pltpu.VMEM((_C, _E), jnp.float32),
            pltpu.VMEM((_C,), jnp.float32),
            pltpu.VMEM((_C,), jnp.float32),
            pltpu.VMEM((_C,), jnp.float32),
            pltpu.VMEM((_C,), jnp.float32),
            pltpu.VMEM((_L,), jnp.float32),
            pltpu.SemaphoreType.DMA,
            pltpu.SemaphoreType.DMA,
            pltpu.SemaphoreType.DMA,
            pltpu.SemaphoreType.DMA,
        ],
    )
    return kfn(iv, jv, yv, fv, w, w_, b, b_)


def kernel(i, j, xij, w, w_, b, b_):
    yv, fv = _pre(xij)
    parts = _sc(i, j, yv, fv, w, w_, b, b_)
    return _post(parts.reshape(4, 128))[0, 0]
